# Initial kernel scaffold; baseline (speedup 1.0000x reference)
import jax
import jax.numpy as jnp
from jax.experimental import pallas as pl


def kernel(e1_batch, e2_batch, adj_shared_batch, adj1_batch, adj2_batch, W_s1, W_s2, W_con, W_dec1, W_dec2, Wq1, Wk1, g1, Wp1, Wq2, Wk2, g2, Wp2, w_omega, u_omega):
    raise NotImplementedError("write your pallas kernel here")


# speedup vs baseline: 3.7863x; 3.7863x over previous
import functools

import jax
import jax.numpy as jnp
from jax import lax
from jax.experimental import pallas as pl
from jax.experimental.pallas import tpu as pltpu
from jax.experimental.pallas import tpu_sc as plsc

N = 10000
D = 128
CH = 128
NPAD = 10240
R = 1000
GRID = N // R
NSUB = 16
ROWS_PT = NPAD // NSUB


def _sc_spmm(table, src, dst, zeros, *, feat_split):
    EP = dst.shape[0]
    per_tile = EP // NSUB if feat_split else EP // (2 * NSUB)
    chunks = per_tile // CH

    mesh = plsc.VectorSubcoreMesh(core_axis_name="c", subcore_axis_name="s")

    @functools.partial(
        pl.kernel,
        out_type=jax.ShapeDtypeStruct((2 * N, D), jnp.float32),
        mesh=mesh,
        scratch_types=[
            pltpu.VMEM((CH,), jnp.int32),
            pltpu.VMEM((CH,), jnp.int32),
            pltpu.VMEM((CH, D), jnp.float32),
            pltpu.VMEM((CH, D), jnp.float32),
            pltpu.VMEM_SHARED((NPAD, D), jnp.float32),
            pltpu.SemaphoreType.DMA,
        ],
    )
    def k(tbl_h, src_h, dst_h, zer_h, out_h, idx_s, idx_d, rows, zbuf,
          shared, sem):
        c = lax.axis_index("c")
        s = lax.axis_index("s")
        pltpu.sync_copy(zer_h, zbuf)
        row0 = s * ROWS_PT
        for j in range(ROWS_PT // CH):
            pltpu.sync_copy(zbuf, shared.at[pl.ds(row0 + j * CH, CH)])
        plsc.subcore_barrier()

        if feat_split:
            src_base = c * EP + s * per_tile
            dst_base = s * per_tile
        else:
            src_base = (c * NSUB + s) * per_tile
            dst_base = src_base

        def step(i, carry):
            so = src_base + i * CH
            do = dst_base + i * CH
            pltpu.sync_copy(src_h.at[pl.ds(so, CH)], idx_s)
            pltpu.async_copy(tbl_h.at[idx_s], rows, sem).wait()
            pltpu.sync_copy(dst_h.at[pl.ds(do, CH)], idx_d)
            pltpu.sync_copy(rows, shared.at[idx_d], add=True)
            return carry

        lax.fori_loop(0, chunks, step, 0)
        plsc.subcore_barrier()

        out0 = c * N + row0
        pltpu.sync_copy(shared.at[pl.ds(row0, 400)],
                        out_h.at[pl.ds(out0, 400)])

        @pl.when(s < NSUB - 1)
        def _():
            pltpu.sync_copy(shared.at[pl.ds(row0 + 400, 240)],
                            out_h.at[pl.ds(out0 + 400, 240)])

    return k(table, src, dst, zeros)


def _tc_supports(e1, e2, Ws1, Ws2, Wc1, Wc2):
    def body(e1_r, e2_r, w1_r, w2_r, wc1_r, wc2_r, s1_r, s2_r, sf_r):
        a = e1_r[...]
        b = e2_r[...]
        s1_r[...] = jnp.dot(a, w1_r[...], preferred_element_type=jnp.float32)
        s2_r[...] = jnp.dot(b, w2_r[...], preferred_element_type=jnp.float32)
        sf_r[...] = (jnp.dot(a, wc1_r[...], preferred_element_type=jnp.float32)
                     + jnp.dot(b, wc2_r[...], preferred_element_type=jnp.float32))

    row = pl.BlockSpec((R, D), lambda i: (i, 0))
    wspec = pl.BlockSpec((D, D), lambda i: (0, 0))
    return pl.pallas_call(
        body,
        grid=(GRID,),
        in_specs=[row, row, wspec, wspec, wspec, wspec],
        out_specs=[row, row, row],
        out_shape=[jax.ShapeDtypeStruct((N, D), jnp.float32)] * 3,
    )(e1, e2, Ws1, Ws2, Wc1, Wc2)


def _rownorm(q):
    return q / (jnp.sqrt(jnp.sum(q * q, axis=-1, keepdims=True)) + 1e-6)


_SQRTD = 128.0 ** 0.5


def _tc_eaa_reduce(fp, Wq1, Wq2, g1r, g2r):
    def body(fp_r, wq1_r, wq2_r, g1_r, g2_r, G1_r, G2_r, acc, st):
        i = pl.program_id(0)

        @pl.when(i == 0)
        def _():
            st[0] = -1e30
            st[1] = 0.0
            st[2] = -1e30
            st[3] = 0.0
            acc[...] = jnp.zeros_like(acc)

        f = fp_r[0] + fp_r[1]

        def accum(wq_r, g_r, mi, si, row):
            q = _rownorm(jnp.dot(f, wq_r[...],
                                 preferred_element_type=jnp.float32))
            qw = jnp.sum(q * g_r[...], axis=-1, keepdims=True) / _SQRTD
            m_old = st[mi]
            m_new = jnp.maximum(m_old, jnp.max(qw))
            w = jnp.exp(qw - m_new)
            scale = jnp.exp(m_old - m_new)
            st[mi] = m_new
            st[si] = st[si] * scale + jnp.sum(w)
            acc[row:row + 1, :] = (acc[row:row + 1, :] * scale
                                   + jnp.sum(w * q, axis=0, keepdims=True))

        accum(wq1_r, g1_r, 0, 1, 0)
        accum(wq2_r, g2_r, 2, 3, 1)

        @pl.when(i == GRID - 1)
        def _():
            G1_r[...] = acc[0:1, :] / st[1]
            G2_r[...] = acc[1:2, :] / st[3]

    row3 = pl.BlockSpec((2, R, D), lambda i: (0, i, 0))
    wspec = pl.BlockSpec((D, D), lambda i: (0, 0))
    gspec = pl.BlockSpec((1, D), lambda i: (0, 0))
    return pl.pallas_call(
        body,
        grid=(GRID,),
        in_specs=[row3, wspec, wspec, gspec, gspec],
        out_specs=[gspec, gspec],
        out_shape=[jax.ShapeDtypeStruct((1, D), jnp.float32)] * 2,
        scratch_shapes=[pltpu.VMEM((8, D), jnp.float32),
                        pltpu.SMEM((4,), jnp.float32)],
    )(fp, Wq1, Wq2, g1r, g2r)


def _tc_fuse(fp, s1p, s2p, Wq1, Wq2, Wk1, Wk2, Wp1, Wp2, G1, G2,
             w_omega, u_omr):
    def body(fp_r, s1p_r, s2p_r, wq1_r, wq2_r, wk1_r, wk2_r, wp1_r, wp2_r,
             G1_r, G2_r, wo_r, uo_r, z1_r, z2_r, z_r, zp_r, al_r):
        f = fp_r[0] + fp_r[1]

        def eaa(wq_r, wk_r, wp_r, G_r, sp_r):
            q = _rownorm(jnp.dot(f, wq_r[...],
                                 preferred_element_type=jnp.float32))
            k = jnp.dot(sp_r[0] + sp_r[1], wk_r[...],
                        preferred_element_type=jnp.float32)
            return jnp.dot(G_r[...] * k, wp_r[...],
                           preferred_element_type=jnp.float32) + q

        s1o = eaa(wq1_r, wk1_r, wp1_r, G1_r, s1p_r)
        s2o = eaa(wq2_r, wk2_r, wp2_r, G2_r, s2p_r)
        z1 = jnp.concatenate([s1o, f], axis=-1)
        z2 = jnp.concatenate([s2o, f], axis=-1)

        wo = wo_r[...]
        uo = uo_r[...]
        vu1 = jnp.sum(jnp.tanh(jnp.dot(z1, wo,
                                       preferred_element_type=jnp.float32))
                      * uo, axis=-1, keepdims=True)
        vu2 = jnp.sum(jnp.tanh(jnp.dot(z2, wo,
                                       preferred_element_type=jnp.float32))
                      * uo, axis=-1, keepdims=True)
        m = jnp.maximum(vu1, vu2)
        ex1 = jnp.exp(vu1 - m)
        ex2 = jnp.exp(vu2 - m)
        tot = ex1 + ex2
        a1 = ex1 / tot
        a2 = ex2 / tot
        z = a1 * z1 + a2 * z2

        z1_r[...] = z1
        z2_r[...] = z2
        z_r[...] = z
        zp_r[0] = z[:, :D]
        zp_r[1] = z[:, D:]
        al_r[...] = jnp.concatenate([a1, a2], axis=1)

    row3 = pl.BlockSpec((2, R, D), lambda i: (0, i, 0))
    w128 = pl.BlockSpec((D, D), lambda i: (0, 0))
    w256 = pl.BlockSpec((2 * D, 2 * D), lambda i: (0, 0))
    g128 = pl.BlockSpec((1, D), lambda i: (0, 0))
    g256 = pl.BlockSpec((1, 2 * D), lambda i: (0, 0))
    row2w = pl.BlockSpec((R, 2 * D), lambda i: (i, 0))
    alspec = pl.BlockSpec((R, 2), lambda i: (i, 0))
    return pl.pallas_call(
        body,
        grid=(GRID,),
        in_specs=[row3, row3, row3, w128, w128, w128, w128, w128, w128,
                  g128, g128, w256, g256],
        out_specs=[row2w, row2w, row2w, row3, alspec],
        out_shape=[
            jax.ShapeDtypeStruct((N, 2 * D), jnp.float32),
            jax.ShapeDtypeStruct((N, 2 * D), jnp.float32),
            jax.ShapeDtypeStruct((N, 2 * D), jnp.float32),
            jax.ShapeDtypeStruct((2, N, D), jnp.float32),
            jax.ShapeDtypeStruct((N, 2), jnp.float32),
        ],
    )(fp, s1p, s2p, Wq1, Wq2, Wk1, Wk2, Wp1, Wp2, G1, G2, w_omega, u_omr)


def _tc_decode(zs, Wd1a, Wd1b, Wd2a, Wd2b):
    def body(zs_r, a1_r, b1_r, a2_r, b2_r, r1_r, r2_r):
        z0 = zs_r[0]
        z1 = zs_r[1]
        r1_r[...] = (jnp.dot(z0, a1_r[...], preferred_element_type=jnp.float32)
                     + jnp.dot(z1, b1_r[...], preferred_element_type=jnp.float32))
        r2_r[...] = (jnp.dot(z0, a2_r[...], preferred_element_type=jnp.float32)
                     + jnp.dot(z1, b2_r[...], preferred_element_type=jnp.float32))

    row3 = pl.BlockSpec((2, R, D), lambda i: (0, i, 0))
    wspec = pl.BlockSpec((D, D), lambda i: (0, 0))
    row = pl.BlockSpec((R, D), lambda i: (i, 0))
    return pl.pallas_call(
        body,
        grid=(GRID,),
        in_specs=[row3, wspec, wspec, wspec, wspec],
        out_specs=[row, row],
        out_shape=[jax.ShapeDtypeStruct((N, D), jnp.float32)] * 2,
    )(zs, Wd1a, Wd1b, Wd2a, Wd2b)


def _pad_edges(adj, ep):
    src = adj[0].astype(jnp.int32)
    dst = adj[1].astype(jnp.int32)
    e = src.shape[0]
    pad = ep - e
    src_p = jnp.concatenate([src, jnp.zeros((pad,), jnp.int32)])
    dst_p = jnp.concatenate([dst, jnp.full((pad,), N, jnp.int32)])
    return src_p, dst_p


def kernel(e1_batch, e2_batch, adj_shared_batch, adj1_batch, adj2_batch,
           W_s1, W_s2, W_con, W_dec1, W_dec2,
           Wq1, Wk1, g1, Wp1, Wq2, Wk2, g2, Wp2,
           w_omega, u_omega):
    e = adj1_batch.shape[1]
    ep = ((e + 2 * NSUB * CH - 1) // (2 * NSUB * CH)) * (2 * NSUB * CH)

    src1, dst1 = _pad_edges(adj1_batch, ep)
    src2, dst2 = _pad_edges(adj2_batch, ep)
    srcs, dsts = _pad_edges(adj_shared_batch, ep)
    srcs2 = jnp.concatenate([srcs, srcs + N])
    zeros = jnp.zeros((CH, D), jnp.float32)

    sup1, sup2, supf = _tc_supports(
        e1_batch, e2_batch, W_s1, W_s2, W_con[:D], W_con[D:])

    s1p = _sc_spmm(sup1, src1, dst1, zeros, feat_split=False).reshape(2, N, D)
    s2p = _sc_spmm(sup2, src2, dst2, zeros, feat_split=False).reshape(2, N, D)
    fp = _sc_spmm(supf, srcs, dsts, zeros, feat_split=False).reshape(2, N, D)

    g1r = g1.reshape(1, D)
    g2r = g2.reshape(1, D)
    G1, G2 = _tc_eaa_reduce(fp, Wq1, Wq2, g1r, g2r)

    z1, z2, z, zpair, alpha = _tc_fuse(
        fp, s1p, s2p, Wq1, Wq2, Wk1, Wk2, Wp1, Wp2, G1, G2,
        w_omega, u_omega.reshape(1, 2 * D))

    zcat = zpair.reshape(2 * N, D)
    zs = _sc_spmm(zcat, srcs2, dsts, zeros, feat_split=True).reshape(2, N, D)

    emb_recon1, emb_recon2 = _tc_decode(
        zs, W_dec1[:D], W_dec1[D:], W_dec2[:D], W_dec2[D:])

    return z1, z2, z, emb_recon1, emb_recon2, alpha[:, :, None]


# speedup vs baseline: 3.8322x; 1.0121x over previous
import functools

import jax
import jax.numpy as jnp
from jax import lax
from jax.experimental import pallas as pl
from jax.experimental.pallas import tpu as pltpu
from jax.experimental.pallas import tpu_sc as plsc

N = 10000
D = 128
CH = 128
NPAD = 10240
R = 1000
GRID = N // R
NSUB = 16
ROWS_PT = NPAD // NSUB


def _sc_spmm(table, src, dst, zeros, *, feat_split):
    chunks = src.shape[0] // (2 * NSUB * CH)

    mesh = plsc.VectorSubcoreMesh(core_axis_name="c", subcore_axis_name="s")

    @functools.partial(
        pl.kernel,
        out_type=jax.ShapeDtypeStruct((2 * N, D), jnp.float32),
        mesh=mesh,
        scratch_types=[
            pltpu.VMEM((CH,), jnp.int32),
            pltpu.VMEM((CH,), jnp.int32),
            pltpu.VMEM((CH, D), jnp.float32),
            pltpu.VMEM_SHARED((NPAD, D), jnp.float32),
            pltpu.SemaphoreType.DMA,
        ],
    )
    def k(tbl_h, src_h, dst_h, zer_h, out_h, src_a, dst_a,
          rows, shared, sg):
        c = lax.axis_index("c")
        s = lax.axis_index("s")
        wid = c * NSUB + s
        src_base = wid * chunks * CH
        dst_base = (s if feat_split else wid) * chunks * CH

        pltpu.sync_copy(zer_h, rows)
        row0 = s * ROWS_PT
        for j in range(ROWS_PT // CH):
            pltpu.sync_copy(rows, shared.at[pl.ds(row0 + j * CH, CH)])
        plsc.subcore_barrier()

        def step(i, carry):
            pltpu.sync_copy(src_h.at[pl.ds(src_base + i * CH, CH)], src_a)
            pltpu.async_copy(tbl_h.at[src_a], rows, sg).wait()
            pltpu.sync_copy(dst_h.at[pl.ds(dst_base + i * CH, CH)], dst_a)
            pltpu.sync_copy(rows, shared.at[dst_a], add=True)
            return carry

        lax.fori_loop(0, chunks, step, 0)
        plsc.subcore_barrier()

        out0 = c * N + row0
        pltpu.sync_copy(shared.at[pl.ds(row0, 400)],
                        out_h.at[pl.ds(out0, 400)])

        @pl.when(s < NSUB - 1)
        def _():
            pltpu.sync_copy(shared.at[pl.ds(row0 + 400, 240)],
                            out_h.at[pl.ds(out0 + 400, 240)])

    return k(table, src, dst, zeros)


def _sc_spmm3(tables, srcs_, dsts_, zeros):
    chunks = srcs_[0].shape[0] // (2 * NSUB * CH)

    mesh = plsc.VectorSubcoreMesh(core_axis_name="c", subcore_axis_name="s")

    @functools.partial(
        pl.kernel,
        out_type=jax.ShapeDtypeStruct((3 * 2 * N, D), jnp.float32),
        mesh=mesh,
        scratch_types=[
            pltpu.VMEM((CH,), jnp.int32),
            pltpu.VMEM((CH,), jnp.int32),
            pltpu.VMEM((CH, D), jnp.float32),
            pltpu.VMEM_SHARED((NPAD, D), jnp.float32),
            pltpu.SemaphoreType.DMA,
        ],
    )
    def k(t1, t2, t3, s1, s2, s3, d1, d2, d3, zer_h, out_h,
          src_a, dst_a, rows, shared, sg):
        c = lax.axis_index("c")
        s = lax.axis_index("s")
        wid = c * NSUB + s
        base = wid * chunks * CH
        row0 = s * ROWS_PT

        for ph, (tbl_h, src_h, dst_h) in enumerate(
                [(t1, s1, d1), (t2, s2, d2), (t3, s3, d3)]):
            pltpu.sync_copy(zer_h, rows)
            for j in range(ROWS_PT // CH):
                pltpu.sync_copy(rows,
                                shared.at[pl.ds(row0 + j * CH, CH)])
            plsc.subcore_barrier()

            def step(i, carry):
                pltpu.sync_copy(src_h.at[pl.ds(base + i * CH, CH)], src_a)
                pltpu.async_copy(tbl_h.at[src_a], rows, sg).wait()
                pltpu.sync_copy(dst_h.at[pl.ds(base + i * CH, CH)], dst_a)
                pltpu.sync_copy(rows, shared.at[dst_a], add=True)
                return carry

            lax.fori_loop(0, chunks, step, 0)
            plsc.subcore_barrier()

            out0 = (2 * ph + c) * N + row0
            pltpu.sync_copy(shared.at[pl.ds(row0, 400)],
                            out_h.at[pl.ds(out0, 400)])

            @pl.when(s < NSUB - 1)
            def _():
                pltpu.sync_copy(shared.at[pl.ds(row0 + 400, 240)],
                                out_h.at[pl.ds(out0 + 400, 240)])

            plsc.subcore_barrier()

    return k(tables[0], tables[1], tables[2],
             srcs_[0], srcs_[1], srcs_[2],
             dsts_[0], dsts_[1], dsts_[2], zeros)


def _tc_supports(e1, e2, Ws1, Ws2, Wc1, Wc2):
    def body(e1_r, e2_r, w1_r, w2_r, wc1_r, wc2_r, s1_r, s2_r, sf_r):
        a = e1_r[...]
        b = e2_r[...]
        s1_r[...] = jnp.dot(a, w1_r[...], preferred_element_type=jnp.float32)
        s2_r[...] = jnp.dot(b, w2_r[...], preferred_element_type=jnp.float32)
        sf_r[...] = (jnp.dot(a, wc1_r[...], preferred_element_type=jnp.float32)
                     + jnp.dot(b, wc2_r[...], preferred_element_type=jnp.float32))

    row = pl.BlockSpec((R, D), lambda i: (i, 0))
    wspec = pl.BlockSpec((D, D), lambda i: (0, 0))
    return pl.pallas_call(
        body,
        grid=(GRID,),
        in_specs=[row, row, wspec, wspec, wspec, wspec],
        out_specs=[row, row, row],
        out_shape=[jax.ShapeDtypeStruct((N, D), jnp.float32)] * 3,
    )(e1, e2, Ws1, Ws2, Wc1, Wc2)


def _rownorm(q):
    return q / (jnp.sqrt(jnp.sum(q * q, axis=-1, keepdims=True)) + 1e-6)


_SQRTD = 128.0 ** 0.5


def _tc_eaa_reduce(fp, Wq1, Wq2, g1r, g2r):
    def body(fp_r, wq1_r, wq2_r, g1_r, g2_r, G1_r, G2_r, acc, st):
        i = pl.program_id(0)

        @pl.when(i == 0)
        def _():
            st[0] = -1e30
            st[1] = 0.0
            st[2] = -1e30
            st[3] = 0.0
            acc[...] = jnp.zeros_like(acc)

        f = fp_r[0] + fp_r[1]

        def accum(wq_r, g_r, mi, si, row):
            q = _rownorm(jnp.dot(f, wq_r[...],
                                 preferred_element_type=jnp.float32))
            qw = jnp.sum(q * g_r[...], axis=-1, keepdims=True) / _SQRTD
            m_old = st[mi]
            m_new = jnp.maximum(m_old, jnp.max(qw))
            w = jnp.exp(qw - m_new)
            scale = jnp.exp(m_old - m_new)
            st[mi] = m_new
            st[si] = st[si] * scale + jnp.sum(w)
            acc[row:row + 1, :] = (acc[row:row + 1, :] * scale
                                   + jnp.sum(w * q, axis=0, keepdims=True))

        accum(wq1_r, g1_r, 0, 1, 0)
        accum(wq2_r, g2_r, 2, 3, 1)

        @pl.when(i == GRID - 1)
        def _():
            G1_r[...] = acc[0:1, :] / st[1]
            G2_r[...] = acc[1:2, :] / st[3]

    row3 = pl.BlockSpec((2, R, D), lambda i: (0, i, 0))
    wspec = pl.BlockSpec((D, D), lambda i: (0, 0))
    gspec = pl.BlockSpec((1, D), lambda i: (0, 0))
    return pl.pallas_call(
        body,
        grid=(GRID,),
        in_specs=[row3, wspec, wspec, gspec, gspec],
        out_specs=[gspec, gspec],
        out_shape=[jax.ShapeDtypeStruct((1, D), jnp.float32)] * 2,
        scratch_shapes=[pltpu.VMEM((8, D), jnp.float32),
                        pltpu.SMEM((4,), jnp.float32)],
    )(fp, Wq1, Wq2, g1r, g2r)


def _tc_fuse(fp, s1p, s2p, Wq1, Wq2, Wk1, Wk2, Wp1, Wp2, G1, G2,
             w_omega, u_omr):
    def body(fp_r, s1p_r, s2p_r, wq1_r, wq2_r, wk1_r, wk2_r, wp1_r, wp2_r,
             G1_r, G2_r, wo_r, uo_r, z1_r, z2_r, z_r, zp_r, al_r):
        f = fp_r[0] + fp_r[1]

        def eaa(wq_r, wk_r, wp_r, G_r, sp_r):
            q = _rownorm(jnp.dot(f, wq_r[...],
                                 preferred_element_type=jnp.float32))
            k = jnp.dot(sp_r[0] + sp_r[1], wk_r[...],
                        preferred_element_type=jnp.float32)
            return jnp.dot(G_r[...] * k, wp_r[...],
                           preferred_element_type=jnp.float32) + q

        s1o = eaa(wq1_r, wk1_r, wp1_r, G1_r, s1p_r)
        s2o = eaa(wq2_r, wk2_r, wp2_r, G2_r, s2p_r)
        z1 = jnp.concatenate([s1o, f], axis=-1)
        z2 = jnp.concatenate([s2o, f], axis=-1)

        wo = wo_r[...]
        uo = uo_r[...]
        vu1 = jnp.sum(jnp.tanh(jnp.dot(z1, wo,
                                       preferred_element_type=jnp.float32))
                      * uo, axis=-1, keepdims=True)
        vu2 = jnp.sum(jnp.tanh(jnp.dot(z2, wo,
                                       preferred_element_type=jnp.float32))
                      * uo, axis=-1, keepdims=True)
        m = jnp.maximum(vu1, vu2)
        ex1 = jnp.exp(vu1 - m)
        ex2 = jnp.exp(vu2 - m)
        tot = ex1 + ex2
        a1 = ex1 / tot
        a2 = ex2 / tot
        z = a1 * z1 + a2 * z2

        z1_r[...] = z1
        z2_r[...] = z2
        z_r[...] = z
        zp_r[0] = z[:, :D]
        zp_r[1] = z[:, D:]
        al_r[...] = jnp.concatenate([a1, a2], axis=1)

    row3 = pl.BlockSpec((2, R, D), lambda i: (0, i, 0))
    w128 = pl.BlockSpec((D, D), lambda i: (0, 0))
    w256 = pl.BlockSpec((2 * D, 2 * D), lambda i: (0, 0))
    g128 = pl.BlockSpec((1, D), lambda i: (0, 0))
    g256 = pl.BlockSpec((1, 2 * D), lambda i: (0, 0))
    row2w = pl.BlockSpec((R, 2 * D), lambda i: (i, 0))
    alspec = pl.BlockSpec((R, 2), lambda i: (i, 0))
    return pl.pallas_call(
        body,
        grid=(GRID,),
        in_specs=[row3, row3, row3, w128, w128, w128, w128, w128, w128,
                  g128, g128, w256, g256],
        out_specs=[row2w, row2w, row2w, row3, alspec],
        out_shape=[
            jax.ShapeDtypeStruct((N, 2 * D), jnp.float32),
            jax.ShapeDtypeStruct((N, 2 * D), jnp.float32),
            jax.ShapeDtypeStruct((N, 2 * D), jnp.float32),
            jax.ShapeDtypeStruct((2, N, D), jnp.float32),
            jax.ShapeDtypeStruct((N, 2), jnp.float32),
        ],
    )(fp, s1p, s2p, Wq1, Wq2, Wk1, Wk2, Wp1, Wp2, G1, G2, w_omega, u_omr)


def _tc_decode(zs, Wd1a, Wd1b, Wd2a, Wd2b):
    def body(zs_r, a1_r, b1_r, a2_r, b2_r, r1_r, r2_r):
        z0 = zs_r[0]
        z1 = zs_r[1]
        r1_r[...] = (jnp.dot(z0, a1_r[...], preferred_element_type=jnp.float32)
                     + jnp.dot(z1, b1_r[...], preferred_element_type=jnp.float32))
        r2_r[...] = (jnp.dot(z0, a2_r[...], preferred_element_type=jnp.float32)
                     + jnp.dot(z1, b2_r[...], preferred_element_type=jnp.float32))

    row3 = pl.BlockSpec((2, R, D), lambda i: (0, i, 0))
    wspec = pl.BlockSpec((D, D), lambda i: (0, 0))
    row = pl.BlockSpec((R, D), lambda i: (i, 0))
    return pl.pallas_call(
        body,
        grid=(GRID,),
        in_specs=[row3, wspec, wspec, wspec, wspec],
        out_specs=[row, row],
        out_shape=[jax.ShapeDtypeStruct((N, D), jnp.float32)] * 2,
    )(zs, Wd1a, Wd1b, Wd2a, Wd2b)


def _pad_edges(adj, ep):
    src = adj[0].astype(jnp.int32)
    dst = adj[1].astype(jnp.int32)
    e = src.shape[0]
    pad = ep - e
    src_p = jnp.concatenate([src, jnp.zeros((pad,), jnp.int32)])
    dst_p = jnp.concatenate([dst, jnp.full((pad,), N, jnp.int32)])
    return src_p, dst_p


def kernel(e1_batch, e2_batch, adj_shared_batch, adj1_batch, adj2_batch,
           W_s1, W_s2, W_con, W_dec1, W_dec2,
           Wq1, Wk1, g1, Wp1, Wq2, Wk2, g2, Wp2,
           w_omega, u_omega):
    e = adj1_batch.shape[1]
    ep = ((e + 2 * NSUB * CH - 1) // (2 * NSUB * CH)) * (2 * NSUB * CH)

    src1, dst1 = _pad_edges(adj1_batch, ep)
    src2, dst2 = _pad_edges(adj2_batch, ep)
    srcs, dsts = _pad_edges(adj_shared_batch, ep)
    srcs2 = jnp.concatenate([srcs, srcs + N])
    dsts2 = dsts
    zeros = jnp.zeros((CH, D), jnp.float32)

    sup1, sup2, supf = _tc_supports(
        e1_batch, e2_batch, W_s1, W_s2, W_con[:D], W_con[D:])

    parts = _sc_spmm3((sup1, sup2, supf), (src1, src2, srcs),
                      (dst1, dst2, dsts), zeros).reshape(3, 2, N, D)
    s1p, s2p, fp = parts[0], parts[1], parts[2]

    g1r = g1.reshape(1, D)
    g2r = g2.reshape(1, D)
    G1, G2 = _tc_eaa_reduce(fp, Wq1, Wq2, g1r, g2r)

    z1, z2, z, zpair, alpha = _tc_fuse(
        fp, s1p, s2p, Wq1, Wq2, Wk1, Wk2, Wp1, Wp2, G1, G2,
        w_omega, u_omega.reshape(1, 2 * D))

    zcat = zpair.reshape(2 * N, D)
    zs = _sc_spmm(zcat, srcs2, dsts2, zeros, feat_split=True).reshape(2, N, D)

    emb_recon1, emb_recon2 = _tc_decode(
        zs, W_dec1[:D], W_dec1[D:], W_dec2[:D], W_dec2[D:])

    return z1, z2, z, emb_recon1, emb_recon2, alpha[:, :, None]
